# Initial kernel scaffold; baseline (speedup 1.0000x reference)
import jax
import jax.numpy as jnp
from jax.experimental import pallas as pl


def kernel(word_ids, entity_ids, prior_probs, word_table, entity_table, att_w, att_b, out_w, out_b):
    raise NotImplementedError("write your pallas kernel here")


# speedup vs baseline: 1.2476x; 1.2476x over previous
import functools

import jax
import jax.numpy as jnp
from jax import lax
from jax.experimental import pallas as pl
from jax.experimental.pallas import tpu as pltpu
from jax.experimental.pallas import tpu_sc as plsc

B = 4096
WLEN = 200
ELEN = 50
DIM = 64
NUM_CLASSES = 16

NC = 2
NS = 16
NW = NC * NS
BPW = B // NW

W_SPLIT = (104, 96)


def _sc_body(word_ids_hbm, entity_ids_hbm, wtab_hbm, etab_hbm,
             wsum_hbm, ent_hbm,
             idx_w, idx_e, rw, ent_v, wsum_v, sem_w, sem_e):
  wid = lax.axis_index("s") * NC + lax.axis_index("c")
  base = wid * BPW

  def item(i, carry):
    b = base + i
    pltpu.sync_copy(word_ids_hbm.at[b], idx_w)
    pltpu.sync_copy(entity_ids_hbm.at[b], idx_e)
    cw0 = pltpu.async_copy(
        wtab_hbm.at[idx_w.at[pl.ds(0, W_SPLIT[0])]],
        rw.at[pl.ds(0, W_SPLIT[0])], sem_w)
    cw1 = pltpu.async_copy(
        wtab_hbm.at[idx_w.at[pl.ds(W_SPLIT[0], W_SPLIT[1])]],
        rw.at[pl.ds(W_SPLIT[0], W_SPLIT[1])], sem_w)
    ce = pltpu.async_copy(etab_hbm.at[idx_e], ent_v, sem_e)
    cw0.wait()
    cw1.wait()

    def red(j, acc):
      return tuple(acc[k] + rw[j, pl.ds(16 * k, 16)] for k in range(4))

    acc = lax.fori_loop(
        0, WLEN, red,
        tuple(jnp.zeros((16,), jnp.float32) for _ in range(4)))
    for k in range(4):
      wsum_v[i, pl.ds(16 * k, 16)] = acc[k]
    ce.wait()
    pltpu.sync_copy(ent_v, ent_hbm.at[b])
    return carry

  lax.fori_loop(0, BPW, item, 0)
  pltpu.sync_copy(wsum_v, wsum_hbm.at[pl.ds(base, BPW)])


@functools.cache
def _sc_gather():
  return pl.kernel(
      _sc_body,
      out_type=(
          jax.ShapeDtypeStruct((B, DIM), jnp.float32),
          jax.ShapeDtypeStruct((B, ELEN, DIM), jnp.float32),
      ),
      mesh=plsc.VectorSubcoreMesh(core_axis_name="c", subcore_axis_name="s"),
      compiler_params=pltpu.CompilerParams(use_tc_tiling_on_sc=False),
      scratch_types=[
          pltpu.VMEM((WLEN,), jnp.int32),
          pltpu.VMEM((ELEN,), jnp.int32),
          pltpu.VMEM((WLEN, DIM), jnp.float32),
          pltpu.VMEM((ELEN, DIM), jnp.float32),
          pltpu.VMEM((BPW, DIM), jnp.float32),
          pltpu.SemaphoreType.DMA,
          pltpu.SemaphoreType.DMA,
      ],
  )


BB = 256


def _tc_body(wids_ref, eids_ref, prior_ref, wsum_ref, ent_ref,
             attw_ref, attb_ref, outw_ref, outb_ref, o_ref):
  wsum = wsum_ref[...]
  ent = ent_ref[...]
  nonzero = jnp.sum((wids_ref[...] != 0).astype(jnp.float32), axis=1,
                    keepdims=True)
  w_norm = jnp.maximum(
      jnp.sqrt(jnp.sum(wsum * wsum, axis=1, keepdims=True)), 1e-12)
  wn = wsum / w_norm
  e_norm = jnp.maximum(
      jnp.sqrt(jnp.sum(ent * ent, axis=2)), 1e-12)
  cos = jnp.sum(wn[:, None, :] * ent, axis=2) / e_norm
  logits = (prior_ref[...] * attw_ref[0, 0] + cos * attw_ref[0, 1]
            + attb_ref[0])
  logits = jnp.where(eids_ref[...] == 0, -1e32, logits)
  m = jnp.max(logits, axis=1, keepdims=True)
  e = jnp.exp(logits - m)
  aw = e / jnp.sum(e, axis=1, keepdims=True)
  feat = jnp.sum(ent * aw[:, :, None], axis=1)
  feat = feat + wsum / nonzero
  o_ref[...] = lax.dot_general(
      feat, outw_ref[...], (((1,), (1,)), ((), ())),
      preferred_element_type=jnp.float32) + outb_ref[...]


def _tc_dense(word_ids, entity_ids, prior_probs, wsum, ent,
              att_w, att_b, out_w, out_b):
  grid = B // BB
  return pl.pallas_call(
      _tc_body,
      grid=(grid,),
      in_specs=[
          pl.BlockSpec((BB, WLEN), lambda i: (i, 0)),
          pl.BlockSpec((BB, ELEN), lambda i: (i, 0)),
          pl.BlockSpec((BB, ELEN), lambda i: (i, 0)),
          pl.BlockSpec((BB, DIM), lambda i: (i, 0)),
          pl.BlockSpec((BB, ELEN, DIM), lambda i: (i, 0, 0)),
          pl.BlockSpec(memory_space=pltpu.SMEM),
          pl.BlockSpec(memory_space=pltpu.SMEM),
          pl.BlockSpec((NUM_CLASSES, DIM), lambda i: (0, 0)),
          pl.BlockSpec((1, NUM_CLASSES), lambda i: (0, 0)),
      ],
      out_specs=pl.BlockSpec((BB, NUM_CLASSES), lambda i: (i, 0)),
      out_shape=jax.ShapeDtypeStruct((B, NUM_CLASSES), jnp.float32),
  )(word_ids, entity_ids, prior_probs, wsum, ent,
    att_w, att_b, out_w, out_b)


def kernel(word_ids, entity_ids, prior_probs, word_table, entity_table,
           att_w, att_b, out_w, out_b):
  wsum, ent = _sc_gather()(word_ids, entity_ids, word_table, entity_table)
  return _tc_dense(word_ids, entity_ids, prior_probs, wsum, ent,
                   att_w, att_b, out_w, out_b.reshape(1, NUM_CLASSES))
